# Initial kernel scaffold; baseline (speedup 1.0000x reference)
import jax
import jax.numpy as jnp
from jax.experimental import pallas as pl


def kernel(x, edge_index, W1, b1, W2, b2):
    raise NotImplementedError("write your pallas kernel here")


# speedup vs baseline: 29.0010x; 29.0010x over previous
import jax
import jax.numpy as jnp
from jax import lax
from jax.experimental import pallas as pl
from jax.experimental.pallas import tpu as pltpu
from jax.experimental.pallas import tpu_sc as plsc

N = 10000
D = 128
NC = 2
NS = 16
B = 128
NB = 79
EW = NB * B
EPAD = NC * NS * EW
NB2 = NC * NB
DH = D // 2
RZ = 632
NACC = NS * RZ
BM = 2000
F32 = jnp.float32


def _mesh():
    return plsc.VectorSubcoreMesh(core_axis_name="c", subcore_axis_name="s")


def _deg_body(dst_hbm, degp_hbm, dst_v, ones_v, zv, deg_sh):
    c = lax.axis_index("c")
    s = lax.axis_index("s")
    soff = pl.multiple_of(s * RZ, 8)
    woff = pl.multiple_of((c * NS + s) * RZ, 8)
    for k in range((RZ + 8) // 16):
        zv[pl.ds(k * 16, 16)] = jnp.zeros((16,), F32)
    pltpu.sync_copy(zv.at[pl.ds(0, RZ)], deg_sh.at[pl.ds(soff, RZ)])
    pltpu.sync_copy(dst_hbm.at[c, s], dst_v)
    for k in range(B // 16):
        ones_v[pl.ds(k * 16, 16)] = jnp.ones((16,), F32)
    plsc.subcore_barrier()

    def body(j, carry):
        pltpu.sync_copy(ones_v, deg_sh.at[dst_v.at[j]], add=True)
        return carry

    lax.fori_loop(0, NB, body, 0)
    plsc.subcore_barrier()
    pltpu.sync_copy(deg_sh.at[pl.ds(soff, RZ)], zv.at[pl.ds(0, RZ)])
    pltpu.sync_copy(zv.at[pl.ds(0, RZ)], degp_hbm.at[pl.ds(woff, RZ)])


def _deg_call(dstw):
    fn = pl.kernel(
        _deg_body,
        out_type=jax.ShapeDtypeStruct((NC * NACC,), F32),
        mesh=_mesh(),
        scratch_types=[
            pltpu.VMEM((NB, B), jnp.int32),
            pltpu.VMEM((B,), F32),
            pltpu.VMEM((RZ + 8,), F32),
            pltpu.VMEM_SHARED((NACC,), F32),
        ],
    )
    return fn(dstw)


def _rows_body(h2_hbm, src2_hbm, dst_hbm, outp_hbm,
               src_v, dst_v, rows_v, zrow_v, cp_v, acc_sh, sem):
    c = lax.axis_index("c")
    s = lax.axis_index("s")
    soff = pl.multiple_of(s * RZ, 8)
    woff = pl.multiple_of((c * NS + s) * RZ, 8)
    for r in range(8):
        for k in range(DH // 16):
            zrow_v[r, pl.ds(k * 16, 16)] = jnp.zeros((16,), F32)

    def zbody(q, carry):
        qoff = pl.multiple_of(soff + q * 8, 8)
        pltpu.sync_copy(zrow_v, acc_sh.at[pl.ds(qoff, 8)])
        return carry

    lax.fori_loop(0, RZ // 8, zbody, 0)
    pltpu.sync_copy(src2_hbm.at[c, s], src_v)
    pltpu.sync_copy(dst_hbm.at[s], dst_v)
    plsc.subcore_barrier()

    def body(j, carry):
        pltpu.async_copy(h2_hbm.at[src_v.at[j]], rows_v, sem).wait()
        pltpu.sync_copy(rows_v, acc_sh.at[dst_v.at[j]], add=True)
        return carry

    lax.fori_loop(0, NB2, body, 0)
    plsc.subcore_barrier()
    for q, sz in ((0, 200), (200, 200), (400, 200), (600, 32)):
        pltpu.sync_copy(acc_sh.at[pl.ds(soff + q, sz)], cp_v.at[pl.ds(0, sz)])
        pltpu.sync_copy(cp_v.at[pl.ds(0, sz)], outp_hbm.at[pl.ds(woff + q, sz)])


def _rows_call(h2, src2w, dstw2):
    fn = pl.kernel(
        _rows_body,
        out_type=jax.ShapeDtypeStruct((NC * NACC, DH), F32),
        mesh=_mesh(),
        scratch_types=[
            pltpu.VMEM((NB2, B), jnp.int32),
            pltpu.VMEM((NB2, B), jnp.int32),
            pltpu.VMEM((B, DH), F32),
            pltpu.VMEM((8, DH), F32),
            pltpu.VMEM((200, DH), F32),
            pltpu.VMEM_SHARED((NACC, DH), F32),
            pltpu.SemaphoreType.DMA,
        ],
        compiler_params=pltpu.CompilerParams(use_tc_tiling_on_sc=False),
    )
    return fn(h2, src2w, dstw2)


def _scal_body(g_hbm, src_hbm, dst_hbm, outp_hbm,
               g_v, src_v, dst_v, vals_v, zv, acc_sh):
    c = lax.axis_index("c")
    s = lax.axis_index("s")
    soff = pl.multiple_of(s * RZ, 8)
    woff = pl.multiple_of((c * NS + s) * RZ, 8)
    for k in range((RZ + 8) // 16):
        zv[pl.ds(k * 16, 16)] = jnp.zeros((16,), F32)
    pltpu.sync_copy(zv.at[pl.ds(0, RZ)], acc_sh.at[pl.ds(soff, RZ)])
    pltpu.sync_copy(g_hbm, g_v)
    pltpu.sync_copy(src_hbm.at[c, s], src_v)
    pltpu.sync_copy(dst_hbm.at[c, s], dst_v)
    plsc.subcore_barrier()

    def body(j, carry):
        for k in range(B // 16):
            idx = src_v[j, k]
            vals_v[pl.ds(k * 16, 16)] = plsc.load_gather(g_v, [idx])
        pltpu.sync_copy(vals_v, acc_sh.at[dst_v.at[j]], add=True)
        return carry

    lax.fori_loop(0, NB, body, 0)
    plsc.subcore_barrier()
    pltpu.sync_copy(acc_sh.at[pl.ds(soff, RZ)], zv.at[pl.ds(0, RZ)])
    pltpu.sync_copy(zv.at[pl.ds(0, RZ)], outp_hbm.at[pl.ds(woff, RZ)])


def _scal_call(g, srcw3, dstw):
    fn = pl.kernel(
        _scal_body,
        out_type=jax.ShapeDtypeStruct((NC * NACC,), F32),
        mesh=_mesh(),
        scratch_types=[
            pltpu.VMEM((N,), F32),
            pltpu.VMEM((NB, B // 16, 16), jnp.int32),
            pltpu.VMEM((NB, B), jnp.int32),
            pltpu.VMEM((B,), F32),
            pltpu.VMEM((RZ + 8,), F32),
            pltpu.VMEM_SHARED((NACC,), F32),
        ],
        compiler_params=pltpu.CompilerParams(needs_layout_passes=False),
    )
    return fn(g, srcw3, dstw)


def _mm1_body(x_ref, w_ref, degp_ref, h_ref, dinv_ref):
    deg = degp_ref[0] + degp_ref[1] + 1.0
    dinv = lax.rsqrt(deg)
    h = jnp.dot(x_ref[...], w_ref[...], preferred_element_type=F32)
    h_ref[...] = h * dinv
    dinv_ref[...] = dinv


def _mm1_call(x, W1, degp3):
    return pl.pallas_call(
        _mm1_body,
        grid=(N // BM,),
        in_specs=[
            pl.BlockSpec((BM, D), lambda i: (i, 0)),
            pl.BlockSpec((D, D), lambda i: (0, 0)),
            pl.BlockSpec((NC, BM, 1), lambda i: (0, i, 0)),
        ],
        out_specs=[
            pl.BlockSpec((BM, D), lambda i: (i, 0)),
            pl.BlockSpec((BM, 1), lambda i: (i, 0)),
        ],
        out_shape=[
            jax.ShapeDtypeStruct((N, D), F32),
            jax.ShapeDtypeStruct((N, 1), F32),
        ],
    )(x, W1, degp3)


def _mm2_body(p_ref, h_ref, dinv_ref, b1_ref, w2_ref, g_ref):
    ps = jnp.concatenate([p_ref[0], p_ref[1]], axis=-1) + h_ref[...]
    o1 = ps * dinv_ref[...] + b1_ref[...]
    r = jnp.maximum(o1, 0.0)
    g = jnp.dot(r, w2_ref[...], preferred_element_type=F32)
    g_ref[...] = g * dinv_ref[...]


def _mm2_call(p, h, dinv, b1r, W2):
    return pl.pallas_call(
        _mm2_body,
        grid=(N // BM,),
        in_specs=[
            pl.BlockSpec((NC, BM, DH), lambda i: (0, i, 0)),
            pl.BlockSpec((BM, D), lambda i: (i, 0)),
            pl.BlockSpec((BM, 1), lambda i: (i, 0)),
            pl.BlockSpec((1, D), lambda i: (0, 0)),
            pl.BlockSpec((D, 1), lambda i: (0, 0)),
        ],
        out_specs=pl.BlockSpec((BM, 1), lambda i: (i, 0)),
        out_shape=jax.ShapeDtypeStruct((N, 1), F32),
    )(p, h, dinv, b1r, W2)


def _fin_body(sp_ref, g_ref, dinv_ref, b2_ref, o_ref):
    acc = sp_ref[0] + sp_ref[1] + g_ref[...]
    o_ref[...] = acc * dinv_ref[...] + b2_ref[...]


def _fin_call(sp3, g, dinv, b2r):
    return pl.pallas_call(
        _fin_body,
        grid=(N // BM,),
        in_specs=[
            pl.BlockSpec((NC, BM, 1), lambda i: (0, i, 0)),
            pl.BlockSpec((BM, 1), lambda i: (i, 0)),
            pl.BlockSpec((BM, 1), lambda i: (i, 0)),
            pl.BlockSpec((1, 1), lambda i: (0, 0)),
        ],
        out_specs=pl.BlockSpec((BM, 1), lambda i: (i, 0)),
        out_shape=jax.ShapeDtypeStruct((N, 1), F32),
    )(sp3, g, dinv, b2r)


def kernel(x, edge_index, W1, b1, W2, b2):
    ei = edge_index.astype(jnp.int32)
    src, dst = ei[0], ei[1]
    pad = EPAD - src.shape[0]
    pid = jnp.arange(pad, dtype=jnp.int32)
    srcp = jnp.concatenate([src, pid % N])
    dstp = jnp.concatenate([dst, N + pid % (NACC - N)])
    srcw3 = srcp.reshape(NC, NS, NB, B // 16, 16)
    dstw = dstp.reshape(NC, NS, NB, B)
    src2w = (2 * srcp).reshape(1, NS, NB2, B) + jnp.arange(
        NC, dtype=jnp.int32).reshape(NC, 1, 1, 1)
    dstw2 = dstp.reshape(NS, NB2, B)
    degp = _deg_call(dstw)
    h, dinv = _mm1_call(x, W1, degp.reshape(NC, NACC, 1))
    p = _rows_call(h.reshape(NC * N, DH), src2w, dstw2)
    g = _mm2_call(p.reshape(NC, NACC, DH), h, dinv,
                  b1.reshape(1, D), W2)
    sp = _scal_call(g.reshape(-1), srcw3, dstw)
    out = _fin_call(sp.reshape(NC, NACC, 1), g, dinv, b2.reshape(1, 1))
    return out.reshape(-1)


# speedup vs baseline: 44.9215x; 1.5490x over previous
import jax
import jax.numpy as jnp
from jax import lax
from jax.experimental import pallas as pl
from jax.experimental.pallas import tpu as pltpu
from jax.experimental.pallas import tpu_sc as plsc

N = 10000
D = 128
NC = 2
NS = 16
B = 128
NB = 80
EW = NB * B
EPAD = NC * NS * EW
NB2 = NC * NB
NBQ = NB2 // 4
DH = D // 2
RZ = 632
NACC = NS * RZ
BM = 2000
F32 = jnp.float32


def _mesh():
    return plsc.VectorSubcoreMesh(core_axis_name="c", subcore_axis_name="s")


def _deg_body(dst_hbm, degp_hbm, dst_v, ones_v, zv, deg_sh):
    c = lax.axis_index("c")
    s = lax.axis_index("s")
    soff = pl.multiple_of(s * RZ, 8)
    woff = pl.multiple_of((c * NS + s) * RZ, 8)
    for k in range((RZ + 8) // 16):
        zv[pl.ds(k * 16, 16)] = jnp.zeros((16,), F32)
    pltpu.sync_copy(zv.at[pl.ds(0, RZ)], deg_sh.at[pl.ds(soff, RZ)])
    pltpu.sync_copy(dst_hbm.at[c, s], dst_v)
    for k in range(B // 16):
        ones_v[pl.ds(k * 16, 16)] = jnp.ones((16,), F32)
    plsc.subcore_barrier()

    def body(j, carry):
        pltpu.sync_copy(ones_v, deg_sh.at[dst_v.at[j]], add=True)
        return carry

    lax.fori_loop(0, NB, body, 0)
    plsc.subcore_barrier()
    pltpu.sync_copy(deg_sh.at[pl.ds(soff, RZ)], zv.at[pl.ds(0, RZ)])
    pltpu.sync_copy(zv.at[pl.ds(0, RZ)], degp_hbm.at[pl.ds(woff, RZ)])


def _deg_call(dstw):
    fn = pl.kernel(
        _deg_body,
        out_type=jax.ShapeDtypeStruct((NC * NACC,), F32),
        mesh=_mesh(),
        scratch_types=[
            pltpu.VMEM((NB, B), jnp.int32),
            pltpu.VMEM((B,), F32),
            pltpu.VMEM((RZ + 8,), F32),
            pltpu.VMEM_SHARED((NACC,), F32),
        ],
    )
    return fn(dstw)


def _rows_body(h2_hbm, src2_hbm, dst_hbm, outp_hbm,
               src_v, dst_v, rows_v, zrow_v, cp_v, acc_sh,
               gs0, gs1, gs2, gs3, ss0, ss1, ss2, ss3):
    gsems = (gs0, gs1, gs2, gs3)
    ssems = (ss0, ss1, ss2, ss3)
    c = lax.axis_index("c")
    s = lax.axis_index("s")
    soff = pl.multiple_of(s * RZ, 8)
    woff = pl.multiple_of((c * NS + s) * RZ, 8)
    for r in range(8):
        for k in range(DH // 16):
            zrow_v[r, pl.ds(k * 16, 16)] = jnp.zeros((16,), F32)

    def zbody(q, carry):
        qoff = pl.multiple_of(soff + q * 8, 8)
        pltpu.sync_copy(zrow_v, acc_sh.at[pl.ds(qoff, 8)])
        return carry

    lax.fori_loop(0, RZ // 8, zbody, 0)
    pltpu.sync_copy(src2_hbm.at[c, s], src_v)
    pltpu.sync_copy(dst_hbm.at[s], dst_v)
    plsc.subcore_barrier()

    for k in range(4):
        pltpu.async_copy(h2_hbm.at[src_v.at[k]], rows_v.at[k], gsems[k])

    def body(q, carry):
        j0 = q * 4
        for k in range(4):
            j = j0 + k
            pltpu.make_async_copy(
                h2_hbm.at[src_v.at[j]], rows_v.at[k], gsems[k]).wait()
            pltpu.async_copy(rows_v.at[k], acc_sh.at[dst_v.at[j]],
                             ssems[k], add=True)

            @pl.when(q < NBQ - 1)
            def _():
                pltpu.make_async_copy(
                    rows_v.at[k], acc_sh.at[dst_v.at[j]], ssems[k]).wait()
                pltpu.async_copy(
                    h2_hbm.at[src_v.at[j + 4]], rows_v.at[k], gsems[k])

        return carry

    lax.fori_loop(0, NBQ, body, 0)
    for k in range(4):
        pltpu.make_async_copy(
            rows_v.at[k], acc_sh.at[dst_v.at[NB2 - 4 + k]], ssems[k]).wait()
    plsc.subcore_barrier()
    for q, sz in ((0, 200), (200, 200), (400, 200), (600, 32)):
        pltpu.sync_copy(acc_sh.at[pl.ds(soff + q, sz)], cp_v.at[pl.ds(0, sz)])
        pltpu.sync_copy(cp_v.at[pl.ds(0, sz)], outp_hbm.at[pl.ds(woff + q, sz)])


def _rows_call(h2, src2w, dstw2):
    fn = pl.kernel(
        _rows_body,
        out_type=jax.ShapeDtypeStruct((NC * NACC, DH), F32),
        mesh=_mesh(),
        scratch_types=[
            pltpu.VMEM((NB2, B), jnp.int32),
            pltpu.VMEM((NB2, B), jnp.int32),
            pltpu.VMEM((4, B, DH), F32),
            pltpu.VMEM((8, DH), F32),
            pltpu.VMEM((200, DH), F32),
            pltpu.VMEM_SHARED((NACC, DH), F32),
        ] + [pltpu.SemaphoreType.DMA] * 8,
        compiler_params=pltpu.CompilerParams(use_tc_tiling_on_sc=False),
    )
    return fn(h2, src2w, dstw2)


def _scal_body(g_hbm, src_hbm, dst_hbm, outp_hbm,
               g_v, src_v, dst_v, vals_v, zv, acc_sh):
    c = lax.axis_index("c")
    s = lax.axis_index("s")
    soff = pl.multiple_of(s * RZ, 8)
    woff = pl.multiple_of((c * NS + s) * RZ, 8)
    for k in range((RZ + 8) // 16):
        zv[pl.ds(k * 16, 16)] = jnp.zeros((16,), F32)
    pltpu.sync_copy(zv.at[pl.ds(0, RZ)], acc_sh.at[pl.ds(soff, RZ)])
    pltpu.sync_copy(g_hbm, g_v)
    pltpu.sync_copy(src_hbm.at[c, s], src_v)
    pltpu.sync_copy(dst_hbm.at[c, s], dst_v)
    plsc.subcore_barrier()

    def body(j, carry):
        for k in range(B // 16):
            idx = src_v[j, k]
            vals_v[pl.ds(k * 16, 16)] = plsc.load_gather(g_v, [idx])
        pltpu.sync_copy(vals_v, acc_sh.at[dst_v.at[j]], add=True)
        return carry

    lax.fori_loop(0, NB, body, 0)
    plsc.subcore_barrier()
    pltpu.sync_copy(acc_sh.at[pl.ds(soff, RZ)], zv.at[pl.ds(0, RZ)])
    pltpu.sync_copy(zv.at[pl.ds(0, RZ)], outp_hbm.at[pl.ds(woff, RZ)])


def _scal_call(g, srcw3, dstw):
    fn = pl.kernel(
        _scal_body,
        out_type=jax.ShapeDtypeStruct((NC * NACC,), F32),
        mesh=_mesh(),
        scratch_types=[
            pltpu.VMEM((N,), F32),
            pltpu.VMEM((NB, B // 16, 16), jnp.int32),
            pltpu.VMEM((NB, B), jnp.int32),
            pltpu.VMEM((B,), F32),
            pltpu.VMEM((RZ + 8,), F32),
            pltpu.VMEM_SHARED((NACC,), F32),
        ],
        compiler_params=pltpu.CompilerParams(needs_layout_passes=False),
    )
    return fn(g, srcw3, dstw)


def _mm1_body(x_ref, w_ref, degp_ref, h_ref, dinv_ref):
    deg = degp_ref[0] + degp_ref[1] + 1.0
    dinv = lax.rsqrt(deg)
    h = jnp.dot(x_ref[...], w_ref[...], preferred_element_type=F32)
    h_ref[...] = h * dinv
    dinv_ref[...] = dinv


def _mm1_call(x, W1, degp3):
    return pl.pallas_call(
        _mm1_body,
        grid=(N // BM,),
        in_specs=[
            pl.BlockSpec((BM, D), lambda i: (i, 0)),
            pl.BlockSpec((D, D), lambda i: (0, 0)),
            pl.BlockSpec((NC, BM, 1), lambda i: (0, i, 0)),
        ],
        out_specs=[
            pl.BlockSpec((BM, D), lambda i: (i, 0)),
            pl.BlockSpec((BM, 1), lambda i: (i, 0)),
        ],
        out_shape=[
            jax.ShapeDtypeStruct((N, D), F32),
            jax.ShapeDtypeStruct((N, 1), F32),
        ],
    )(x, W1, degp3)


def _mm2_body(p_ref, h_ref, dinv_ref, b1_ref, w2_ref, g_ref):
    ps = jnp.concatenate([p_ref[0], p_ref[1]], axis=-1) + h_ref[...]
    o1 = ps * dinv_ref[...] + b1_ref[...]
    r = jnp.maximum(o1, 0.0)
    g = jnp.dot(r, w2_ref[...], preferred_element_type=F32)
    g_ref[...] = g * dinv_ref[...]


def _mm2_call(p, h, dinv, b1r, W2):
    return pl.pallas_call(
        _mm2_body,
        grid=(N // BM,),
        in_specs=[
            pl.BlockSpec((NC, BM, DH), lambda i: (0, i, 0)),
            pl.BlockSpec((BM, D), lambda i: (i, 0)),
            pl.BlockSpec((BM, 1), lambda i: (i, 0)),
            pl.BlockSpec((1, D), lambda i: (0, 0)),
            pl.BlockSpec((D, 1), lambda i: (0, 0)),
        ],
        out_specs=pl.BlockSpec((BM, 1), lambda i: (i, 0)),
        out_shape=jax.ShapeDtypeStruct((N, 1), F32),
    )(p, h, dinv, b1r, W2)


def _fin_body(sp_ref, g_ref, dinv_ref, b2_ref, o_ref):
    acc = sp_ref[0] + sp_ref[1] + g_ref[...]
    o_ref[...] = acc * dinv_ref[...] + b2_ref[...]


def _fin_call(sp3, g, dinv, b2r):
    return pl.pallas_call(
        _fin_body,
        grid=(N // BM,),
        in_specs=[
            pl.BlockSpec((NC, BM, 1), lambda i: (0, i, 0)),
            pl.BlockSpec((BM, 1), lambda i: (i, 0)),
            pl.BlockSpec((BM, 1), lambda i: (i, 0)),
            pl.BlockSpec((1, 1), lambda i: (0, 0)),
        ],
        out_specs=pl.BlockSpec((BM, 1), lambda i: (i, 0)),
        out_shape=jax.ShapeDtypeStruct((N, 1), F32),
    )(sp3, g, dinv, b2r)


def kernel(x, edge_index, W1, b1, W2, b2):
    ei = edge_index.astype(jnp.int32)
    src, dst = ei[0], ei[1]
    pad = EPAD - src.shape[0]
    pid = jnp.arange(pad, dtype=jnp.int32)
    srcp = jnp.concatenate([src, pid % N])
    dstp = jnp.concatenate([dst, N + pid % (NACC - N)])
    srcw3 = srcp.reshape(NC, NS, NB, B // 16, 16)
    dstw = dstp.reshape(NC, NS, NB, B)
    src2w = (2 * srcp).reshape(1, NS, NB2, B) + jnp.arange(
        NC, dtype=jnp.int32).reshape(NC, 1, 1, 1)
    dstw2 = dstp.reshape(NS, NB2, B)
    degp = _deg_call(dstw)
    h, dinv = _mm1_call(x, W1, degp.reshape(NC, NACC, 1))
    p = _rows_call(h.reshape(NC * N, DH), src2w, dstw2)
    g = _mm2_call(p.reshape(NC, NACC, DH), h, dinv,
                  b1.reshape(1, D), W2)
    sp = _scal_call(g.reshape(-1), srcw3, dstw)
    out = _fin_call(sp.reshape(NC, NACC, 1), g, dinv, b2.reshape(1, 1))
    return out.reshape(-1)


# speedup vs baseline: 52.1504x; 1.1609x over previous
import jax
import jax.numpy as jnp
from jax import lax
from jax.experimental import pallas as pl
from jax.experimental.pallas import tpu as pltpu
from jax.experimental.pallas import tpu_sc as plsc

N = 10000
D = 128
NC = 2
NS = 16
B = 128
NB = 80
EW = NB * B
EPAD = NC * NS * EW
NB2 = NC * NB
ND = 4
NBQ = NB2 // ND
DH = D // 2
RZ = 632
NACC = NS * RZ
BM = 2000
EB = EPAD // (N // BM)
F32 = jnp.float32


def _mesh():
    return plsc.VectorSubcoreMesh(core_axis_name="c", subcore_axis_name="s")


def _deg_body(dst_hbm, degp_hbm, dst_v, ones_v, zv, deg_sh, sem):
    c = lax.axis_index("c")
    s = lax.axis_index("s")
    soff = pl.multiple_of(s * RZ, 8)
    woff = pl.multiple_of((c * NS + s) * RZ, 8)
    for k in range((RZ + 8) // 16):
        zv[pl.ds(k * 16, 16)] = jnp.zeros((16,), F32)
    pltpu.sync_copy(zv.at[pl.ds(0, RZ)], deg_sh.at[pl.ds(soff, RZ)])
    pltpu.sync_copy(dst_hbm.at[c, s], dst_v)
    for k in range(B // 16):
        ones_v[pl.ds(k * 16, 16)] = jnp.ones((16,), F32)
    plsc.subcore_barrier()

    def body(j8, carry):
        j0 = j8 * 8
        for k in range(8):
            pltpu.async_copy(ones_v, deg_sh.at[dst_v.at[j0 + k]], sem,
                             add=True)
        for k in range(8):
            pltpu.make_async_copy(
                ones_v, deg_sh.at[dst_v.at[j0 + k]], sem).wait()
        return carry

    lax.fori_loop(0, NB // 8, body, 0)
    plsc.subcore_barrier()
    pltpu.sync_copy(deg_sh.at[pl.ds(soff, RZ)], zv.at[pl.ds(0, RZ)])
    pltpu.sync_copy(zv.at[pl.ds(0, RZ)], degp_hbm.at[pl.ds(woff, RZ)])


def _deg_call(dstw):
    fn = pl.kernel(
        _deg_body,
        out_type=jax.ShapeDtypeStruct((NC * NACC,), F32),
        mesh=_mesh(),
        scratch_types=[
            pltpu.VMEM((NB, B), jnp.int32),
            pltpu.VMEM((B,), F32),
            pltpu.VMEM((RZ + 8,), F32),
            pltpu.VMEM_SHARED((NACC,), F32),
            pltpu.SemaphoreType.DMA,
        ],
    )
    return fn(dstw)


def _rows_body(h2_hbm, src2_hbm, dst_hbm, outp_hbm,
               src_v, dst_v, rows_v, cp_v, acc_sh, *sems):
    gsems = sems[:ND]
    ssems = sems[ND:]
    c = lax.axis_index("c")
    s = lax.axis_index("s")
    soff = pl.multiple_of(s * RZ, 8)
    woff = pl.multiple_of((c * NS + s) * RZ, 8)

    def zfill(r, carry):
        for k in range(DH // 16):
            cp_v[r, pl.ds(k * 16, 16)] = jnp.zeros((16,), F32)
        return carry

    lax.fori_loop(0, 200, zfill, 0)
    for q, sz in ((0, 200), (200, 200), (400, 200), (600, 32)):
        pltpu.sync_copy(cp_v.at[pl.ds(0, sz)],
                        acc_sh.at[pl.ds(soff + q, sz)])
    pltpu.sync_copy(src2_hbm.at[c, s], src_v)
    pltpu.sync_copy(dst_hbm.at[s], dst_v)
    plsc.subcore_barrier()

    for k in range(ND):
        pltpu.async_copy(h2_hbm.at[src_v.at[k]], rows_v.at[k], gsems[k])

    def body(q, carry):
        j0 = q * ND
        for k in range(ND):
            j = j0 + k
            pltpu.make_async_copy(
                h2_hbm.at[src_v.at[j]], rows_v.at[k], gsems[k]).wait()
            pltpu.async_copy(rows_v.at[k], acc_sh.at[dst_v.at[j]],
                             ssems[k], add=True)

            @pl.when(q < NBQ - 1)
            def _():
                pltpu.make_async_copy(
                    rows_v.at[k], acc_sh.at[dst_v.at[j]], ssems[k]).wait()
                pltpu.async_copy(
                    h2_hbm.at[src_v.at[j + ND]], rows_v.at[k], gsems[k])

        return carry

    lax.fori_loop(0, NBQ, body, 0)
    for k in range(ND):
        pltpu.make_async_copy(
            rows_v.at[k], acc_sh.at[dst_v.at[NB2 - ND + k]], ssems[k]).wait()
    plsc.subcore_barrier()
    for q, sz in ((0, 200), (200, 200), (400, 200), (600, 32)):
        pltpu.sync_copy(acc_sh.at[pl.ds(soff + q, sz)], cp_v.at[pl.ds(0, sz)])
        pltpu.sync_copy(cp_v.at[pl.ds(0, sz)], outp_hbm.at[pl.ds(woff + q, sz)])


def _rows_call(h2, src2w, dstw2):
    fn = pl.kernel(
        _rows_body,
        out_type=jax.ShapeDtypeStruct((NC * NACC, DH), F32),
        mesh=_mesh(),
        scratch_types=[
            pltpu.VMEM((NB2, B), jnp.int32),
            pltpu.VMEM((NB2, B), jnp.int32),
            pltpu.VMEM((ND, B, DH), F32),
            pltpu.VMEM((200, DH), F32),
            pltpu.VMEM_SHARED((NACC, DH), F32),
        ] + [pltpu.SemaphoreType.DMA] * (2 * ND),
        compiler_params=pltpu.CompilerParams(use_tc_tiling_on_sc=False),
    )
    return fn(h2, src2w, dstw2)


def _scal_body(g_hbm, dinv_hbm, b2_hbm, src_hbm, dst_hbm, out_hbm,
               g_v, dinv_v, b2_v, src_v, dst_v, vals_v, zv, acc_sh,
               sem0, sem1):
    c = lax.axis_index("c")
    s = lax.axis_index("s")

    @pl.when(c == 0)
    def _():
        soff = pl.multiple_of(s * RZ, 8)
        for k in range((RZ + 8) // 16):
            zv[pl.ds(k * 16, 16)] = jnp.zeros((16,), F32)
        pltpu.sync_copy(zv.at[pl.ds(0, RZ)], acc_sh.at[pl.ds(soff, RZ)])
        pltpu.sync_copy(g_hbm, g_v)
        pltpu.sync_copy(dinv_hbm, dinv_v)
        pltpu.sync_copy(b2_hbm, b2_v)
        pltpu.sync_copy(src_hbm.at[s], src_v)
        pltpu.sync_copy(dst_hbm.at[s], dst_v)
        plsc.subcore_barrier()

        def body(q, carry):
            j0 = q * 2
            for par in range(2):
                j = j0 + par
                vb, sm = (vals_v.at[par],
                          sem0 if par == 0 else sem1)

                @pl.when(q > 0)
                def _():
                    pltpu.make_async_copy(
                        vb, acc_sh.at[dst_v.at[j - 2]], sm).wait()

                for k in range(B // 16):
                    idx = src_v[j, pl.ds(k * 16, 16)]
                    vb[pl.ds(k * 16, 16)] = plsc.load_gather(g_v, [idx])
                pltpu.async_copy(vb, acc_sh.at[dst_v.at[j]], sm, add=True)
            return carry

        lax.fori_loop(0, NB2 // 2, body, 0)
        for par in range(2):
            sm = sem0 if par == 0 else sem1
            pltpu.make_async_copy(
                vals_v.at[par], acc_sh.at[dst_v.at[NB2 - 2 + par]], sm).wait()
        plsc.subcore_barrier()

        pltpu.sync_copy(acc_sh.at[pl.ds(soff, RZ)], zv.at[pl.ds(0, RZ)])
        b2s = b2_v[pl.ds(0, 16)]
        for k in range((RZ + 8) // 16):
            o = pl.ds(k * 16, 16)
            so = pl.ds(soff + k * 16, 16)
            zv[o] = dinv_v[so] * (zv[o] + g_v[so]) + b2s
        pltpu.sync_copy(zv.at[pl.ds(0, RZ)], out_hbm.at[pl.ds(soff, RZ)])


def _scal_call(g, dinv, b2w, srcw2, dstw2):
    fn = pl.kernel(
        _scal_body,
        out_type=jax.ShapeDtypeStruct((NACC,), F32),
        mesh=_mesh(),
        scratch_types=[
            pltpu.VMEM((NACC + 16,), F32),
            pltpu.VMEM((NACC + 16,), F32),
            pltpu.VMEM((16,), F32),
            pltpu.VMEM((NB2, B), jnp.int32),
            pltpu.VMEM((NB2, B), jnp.int32),
            pltpu.VMEM((2, B), F32),
            pltpu.VMEM((RZ + 8,), F32),
            pltpu.VMEM_SHARED((NACC,), F32),
            pltpu.SemaphoreType.DMA,
            pltpu.SemaphoreType.DMA,
        ],
        compiler_params=pltpu.CompilerParams(needs_layout_passes=False),
    )
    return fn(g, dinv, b2w, srcw2, dstw2)


def _mm1_body(x_ref, w_ref, degp_ref, sp_ref, h_ref, dinv_ref, s2_ref):
    deg = degp_ref[0] + degp_ref[1] + 1.0
    dinv = lax.rsqrt(deg)
    h = jnp.dot(x_ref[...], w_ref[...], preferred_element_type=F32)
    h_ref[...] = h * dinv
    dinv_ref[...] = dinv
    two_src = 2 * sp_ref[...]
    cix = lax.broadcasted_iota(jnp.int32, (NC,) + two_src.shape, 0)
    s2_ref[...] = two_src[None] + cix


def _mm1_call(x, W1, degp3, srcp2):
    return pl.pallas_call(
        _mm1_body,
        grid=(N // BM,),
        in_specs=[
            pl.BlockSpec((BM, D), lambda i: (i, 0)),
            pl.BlockSpec((D, D), lambda i: (0, 0)),
            pl.BlockSpec((NC, BM, 1), lambda i: (0, i, 0)),
            pl.BlockSpec((EB // B, B), lambda i: (i, 0)),
        ],
        out_specs=[
            pl.BlockSpec((BM, D), lambda i: (i, 0)),
            pl.BlockSpec((BM, 1), lambda i: (i, 0)),
            pl.BlockSpec((NC, EB // B, B), lambda i: (0, i, 0)),
        ],
        out_shape=[
            jax.ShapeDtypeStruct((N, D), F32),
            jax.ShapeDtypeStruct((N, 1), F32),
            jax.ShapeDtypeStruct((NC, EPAD // B, B), jnp.int32),
        ],
    )(x, W1, degp3, srcp2)


def _mm2_body(p_ref, h_ref, dinv_ref, b1_ref, w2_ref, g_ref):
    ps = jnp.concatenate([p_ref[0], p_ref[1]], axis=-1) + h_ref[...]
    o1 = ps * dinv_ref[...] + b1_ref[...]
    r = jnp.maximum(o1, 0.0)
    g = jnp.dot(r, w2_ref[...], preferred_element_type=F32)
    g_ref[...] = g * dinv_ref[...]


def _mm2_call(p, h, dinv, b1r, W2):
    return pl.pallas_call(
        _mm2_body,
        grid=(N // BM,),
        in_specs=[
            pl.BlockSpec((NC, BM, DH), lambda i: (0, i, 0)),
            pl.BlockSpec((BM, D), lambda i: (i, 0)),
            pl.BlockSpec((BM, 1), lambda i: (i, 0)),
            pl.BlockSpec((1, D), lambda i: (0, 0)),
            pl.BlockSpec((D, 1), lambda i: (0, 0)),
        ],
        out_specs=pl.BlockSpec((BM, 1), lambda i: (i, 0)),
        out_shape=jax.ShapeDtypeStruct((N, 1), F32),
    )(p, h, dinv, b1r, W2)


def kernel(x, edge_index, W1, b1, W2, b2):
    ei = edge_index.astype(jnp.int32)
    src, dst = ei[0], ei[1]
    pad = EPAD - src.shape[0]
    pid = jnp.arange(pad, dtype=jnp.int32)
    srcp = jnp.concatenate([src, pid % N])
    dstp = jnp.concatenate([dst, N + pid % (NACC - N)])
    srcw2 = srcp.reshape(NS, NB2, B)
    dstw = dstp.reshape(NC, NS, NB, B)
    dstw2 = dstp.reshape(NS, NB2, B)

    degp = _deg_call(dstw)
    h, dinv, src2w = _mm1_call(x, W1, degp.reshape(NC, NACC, 1),
                               srcp.reshape(EPAD // B, B))
    p = _rows_call(h.reshape(NC * N, DH),
                   src2w.reshape(NC, NS, NB2, B), dstw2)
    g = _mm2_call(p.reshape(NC, NACC, DH), h, dinv,
                  b1.reshape(1, D), W2)
    gp = jnp.pad(g.reshape(-1), (0, NACC + 16 - N))
    dinvp = jnp.pad(dinv.reshape(-1), (0, NACC + 16 - N))
    b2w = jnp.broadcast_to(b2, (16,))
    out = _scal_call(gp, dinvp, b2w, srcw2, dstw2)
    return out[:N]


# speedup vs baseline: 53.1533x; 1.0192x over previous
import jax
import jax.numpy as jnp
from jax import lax
from jax.experimental import pallas as pl
from jax.experimental.pallas import tpu as pltpu
from jax.experimental.pallas import tpu_sc as plsc

N = 10000
D = 128
NC = 2
NS = 16
B = 128
NB = 80
EW = NB * B
EPAD = NC * NS * EW
NB2 = NC * NB
ND = 4
NBQ = NB2 // ND
DH = D // 2
RZ = 632
NACC = NS * RZ
BM = 2000
EB = EPAD // (N // BM)
F32 = jnp.float32


def _mesh():
    return plsc.VectorSubcoreMesh(core_axis_name="c", subcore_axis_name="s")


def _deg_body(dst_hbm, degp_hbm, dst_v, ones_v, zv, deg_sh, sem):
    c = lax.axis_index("c")
    s = lax.axis_index("s")
    soff = pl.multiple_of(s * RZ, 8)
    woff = pl.multiple_of((c * NS + s) * RZ, 8)
    for k in range((RZ + 8) // 16):
        zv[pl.ds(k * 16, 16)] = jnp.zeros((16,), F32)
    pltpu.sync_copy(zv.at[pl.ds(0, RZ)], deg_sh.at[pl.ds(soff, RZ)])
    pltpu.sync_copy(dst_hbm.at[c, s], dst_v)
    for k in range(B // 16):
        ones_v[pl.ds(k * 16, 16)] = jnp.ones((16,), F32)
    plsc.subcore_barrier()

    def body(j8, carry):
        j0 = j8 * 8
        for k in range(8):
            pltpu.async_copy(ones_v, deg_sh.at[dst_v.at[j0 + k]], sem,
                             add=True)
        for k in range(8):
            pltpu.make_async_copy(
                ones_v, deg_sh.at[dst_v.at[j0 + k]], sem).wait()
        return carry

    lax.fori_loop(0, NB // 8, body, 0)
    plsc.subcore_barrier()
    pltpu.sync_copy(deg_sh.at[pl.ds(soff, RZ)], zv.at[pl.ds(0, RZ)])
    pltpu.sync_copy(zv.at[pl.ds(0, RZ)], degp_hbm.at[pl.ds(woff, RZ)])


def _deg_call(dstw):
    fn = pl.kernel(
        _deg_body,
        out_type=jax.ShapeDtypeStruct((NC * NACC,), F32),
        mesh=_mesh(),
        scratch_types=[
            pltpu.VMEM((NB, B), jnp.int32),
            pltpu.VMEM((B,), F32),
            pltpu.VMEM((RZ + 8,), F32),
            pltpu.VMEM_SHARED((NACC,), F32),
            pltpu.SemaphoreType.DMA,
        ],
    )
    return fn(dstw)


def _rows_body(h2_hbm, src2_hbm, dst_hbm, outp_hbm,
               src_v, dst_v, rows_v, cp_v, acc_sh, *sems):
    gsems = sems[:ND]
    ssems = sems[ND:]
    c = lax.axis_index("c")
    s = lax.axis_index("s")
    soff = pl.multiple_of(s * RZ, 8)
    woff = pl.multiple_of((c * NS + s) * RZ, 8)

    def zfill(r, carry):
        for k in range(DH // 16):
            cp_v[r, pl.ds(k * 16, 16)] = jnp.zeros((16,), F32)
        return carry

    lax.fori_loop(0, 200, zfill, 0)
    for q, sz in ((0, 200), (200, 200), (400, 200), (600, 32)):
        pltpu.sync_copy(cp_v.at[pl.ds(0, sz)],
                        acc_sh.at[pl.ds(soff + q, sz)])
    pltpu.sync_copy(src2_hbm.at[s], src_v)
    pltpu.sync_copy(dst_hbm.at[s], dst_v)
    plsc.subcore_barrier()

    cvec = jnp.zeros((16,), jnp.int32) + c

    def to_table_idx(j):
        for t in range(B // 16):
            sl = pl.ds(t * 16, 16)
            src_v[j, sl] = 2 * src_v[j, sl] + cvec

    for k in range(ND):
        to_table_idx(k)
        pltpu.async_copy(h2_hbm.at[src_v.at[k]], rows_v.at[k], gsems[k])

    def body(q, carry):
        j0 = q * ND
        for k in range(ND):
            j = j0 + k
            pltpu.make_async_copy(
                h2_hbm.at[src_v.at[j]], rows_v.at[k], gsems[k]).wait()
            pltpu.async_copy(rows_v.at[k], acc_sh.at[dst_v.at[j]],
                             ssems[k], add=True)

            @pl.when(q < NBQ - 1)
            def _():
                to_table_idx(j + ND)
                pltpu.make_async_copy(
                    rows_v.at[k], acc_sh.at[dst_v.at[j]], ssems[k]).wait()
                pltpu.async_copy(
                    h2_hbm.at[src_v.at[j + ND]], rows_v.at[k], gsems[k])

        return carry

    lax.fori_loop(0, NBQ, body, 0)
    for k in range(ND):
        pltpu.make_async_copy(
            rows_v.at[k], acc_sh.at[dst_v.at[NB2 - ND + k]], ssems[k]).wait()
    plsc.subcore_barrier()
    for q, sz in ((0, 200), (200, 200), (400, 200), (600, 32)):
        pltpu.sync_copy(acc_sh.at[pl.ds(soff + q, sz)], cp_v.at[pl.ds(0, sz)])
        pltpu.sync_copy(cp_v.at[pl.ds(0, sz)], outp_hbm.at[pl.ds(woff + q, sz)])


def _rows_call(h2, src2w, dstw2):
    fn = pl.kernel(
        _rows_body,
        out_type=jax.ShapeDtypeStruct((NC * NACC, DH), F32),
        mesh=_mesh(),
        scratch_types=[
            pltpu.VMEM((NB2, B), jnp.int32),
            pltpu.VMEM((NB2, B), jnp.int32),
            pltpu.VMEM((ND, B, DH), F32),
            pltpu.VMEM((200, DH), F32),
            pltpu.VMEM_SHARED((NACC, DH), F32),
        ] + [pltpu.SemaphoreType.DMA] * (2 * ND),
        compiler_params=pltpu.CompilerParams(use_tc_tiling_on_sc=False,
                                             needs_layout_passes=False),
    )
    return fn(h2, src2w, dstw2)


def _scal_body(g_hbm, dinv_hbm, b2_hbm, src_hbm, dst_hbm, out_hbm,
               g_v, dinv_v, b2_v, src_v, dst_v, vals_v, zv, acc_sh,
               g_sh, *sems):
    c = lax.axis_index("c")
    s = lax.axis_index("s")

    @pl.when(c == 0)
    def _():
        soff = pl.multiple_of(s * RZ, 8)
        for k in range((RZ + 8) // 16):
            zv[pl.ds(k * 16, 16)] = jnp.zeros((16,), F32)
        pltpu.sync_copy(zv.at[pl.ds(0, RZ)], acc_sh.at[pl.ds(soff, RZ)])
        pltpu.sync_copy(g_hbm.at[pl.ds(soff, RZ)], zv.at[pl.ds(0, RZ)])
        pltpu.sync_copy(zv.at[pl.ds(0, RZ)], g_sh.at[pl.ds(soff, RZ)])

        @pl.when(s == 0)
        def _():
            pltpu.sync_copy(g_hbm.at[pl.ds(NACC, 16)], b2_v)
            pltpu.sync_copy(b2_v, g_sh.at[pl.ds(NACC, 16)])

        pltpu.sync_copy(dinv_hbm.at[pl.ds(soff, RZ + 16)], dinv_v)
        pltpu.sync_copy(b2_hbm, b2_v)
        pltpu.sync_copy(src_hbm.at[s], src_v)
        pltpu.sync_copy(dst_hbm.at[s], dst_v)
        plsc.subcore_barrier()
        pltpu.sync_copy(g_sh, g_v)

        def body(q, carry):
            j0 = q * 8
            for par in range(8):
                j = j0 + par
                vb = vals_v.at[par]

                @pl.when(q > 0)
                def _():
                    pltpu.make_async_copy(
                        vb, acc_sh.at[dst_v.at[j - 8]], sems[par]).wait()

                for k in range(B // 16):
                    idx = src_v[j, pl.ds(k * 16, 16)]
                    vb[pl.ds(k * 16, 16)] = plsc.load_gather(g_v, [idx])
                pltpu.async_copy(vb, acc_sh.at[dst_v.at[j]], sems[par],
                                 add=True)
            return carry

        lax.fori_loop(0, NB2 // 8, body, 0)
        for par in range(8):
            pltpu.make_async_copy(
                vals_v.at[par], acc_sh.at[dst_v.at[NB2 - 8 + par]],
                sems[par]).wait()
        plsc.subcore_barrier()

        pltpu.sync_copy(acc_sh.at[pl.ds(soff, RZ)], zv.at[pl.ds(0, RZ)])
        b2s = b2_v[pl.ds(0, 16)]
        for k in range((RZ + 8) // 16):
            o = pl.ds(k * 16, 16)
            so = pl.ds(soff + k * 16, 16)
            zv[o] = dinv_v[o] * (zv[o] + g_v[so]) + b2s
        pltpu.sync_copy(zv.at[pl.ds(0, RZ)], out_hbm.at[pl.ds(soff, RZ)])


def _scal_call(g, dinv, b2w, srcw2, dstw2):
    fn = pl.kernel(
        _scal_body,
        out_type=jax.ShapeDtypeStruct((NACC,), F32),
        mesh=_mesh(),
        scratch_types=[
            pltpu.VMEM((NACC + 16,), F32),
            pltpu.VMEM((RZ + 16,), F32),
            pltpu.VMEM((16,), F32),
            pltpu.VMEM((NB2, B), jnp.int32),
            pltpu.VMEM((NB2, B), jnp.int32),
            pltpu.VMEM((8, B), F32),
            pltpu.VMEM((RZ + 8,), F32),
            pltpu.VMEM_SHARED((NACC,), F32),
            pltpu.VMEM_SHARED((NACC + 16,), F32),
        ] + [pltpu.SemaphoreType.DMA] * 8,
        compiler_params=pltpu.CompilerParams(needs_layout_passes=False),
    )
    return fn(g, dinv, b2w, srcw2, dstw2)


def _mm1_body(x_ref, w_ref, degp_ref, h_ref, dinv_ref):
    deg = degp_ref[0] + degp_ref[1] + 1.0
    dinv = lax.rsqrt(deg)
    h = jnp.dot(x_ref[...], w_ref[...], preferred_element_type=F32)
    h_ref[...] = h * dinv
    dinv_ref[...] = dinv


def _mm1_call(x, W1, degp3):
    return pl.pallas_call(
        _mm1_body,
        grid=(N // BM,),
        in_specs=[
            pl.BlockSpec((BM, D), lambda i: (i, 0)),
            pl.BlockSpec((D, D), lambda i: (0, 0)),
            pl.BlockSpec((NC, BM, 1), lambda i: (0, i, 0)),
        ],
        out_specs=[
            pl.BlockSpec((BM, D), lambda i: (i, 0)),
            pl.BlockSpec((BM, 1), lambda i: (i, 0)),
        ],
        out_shape=[
            jax.ShapeDtypeStruct((N, D), F32),
            jax.ShapeDtypeStruct((N, 1), F32),
        ],
    )(x, W1, degp3)


def _mm2_body(p_ref, h_ref, dinv_ref, b1_ref, w2_ref, g_ref):
    ps = jnp.concatenate([p_ref[0], p_ref[1]], axis=-1) + h_ref[...]
    o1 = ps * dinv_ref[...] + b1_ref[...]
    r = jnp.maximum(o1, 0.0)
    g = jnp.dot(r, w2_ref[...], preferred_element_type=F32)
    g_ref[...] = g * dinv_ref[...]


def _mm2_call(p, h, dinv, b1r, W2):
    return pl.pallas_call(
        _mm2_body,
        grid=(N // BM,),
        in_specs=[
            pl.BlockSpec((NC, BM, DH), lambda i: (0, i, 0)),
            pl.BlockSpec((BM, D), lambda i: (i, 0)),
            pl.BlockSpec((BM, 1), lambda i: (i, 0)),
            pl.BlockSpec((1, D), lambda i: (0, 0)),
            pl.BlockSpec((D, 1), lambda i: (0, 0)),
        ],
        out_specs=pl.BlockSpec((BM, 1), lambda i: (i, 0)),
        out_shape=jax.ShapeDtypeStruct((N, 1), F32),
    )(p, h, dinv, b1r, W2)


def kernel(x, edge_index, W1, b1, W2, b2):
    ei = edge_index.astype(jnp.int32)
    src, dst = ei[0], ei[1]
    pad = EPAD - src.shape[0]
    pid = jnp.arange(pad, dtype=jnp.int32)
    srcp = jnp.concatenate([src, pid % N])
    dstp = jnp.concatenate([dst, N + pid % (NACC - N)])
    srcw2 = srcp.reshape(NS, NB2, B)
    dstw = dstp.reshape(NC, NS, NB, B)
    dstw2 = dstp.reshape(NS, NB2, B)

    degp = _deg_call(dstw)
    h, dinv = _mm1_call(x, W1, degp.reshape(NC, NACC, 1))
    p = _rows_call(h.reshape(NC * N, DH), srcw2, dstw2)
    g = _mm2_call(p.reshape(NC, NACC, DH), h, dinv,
                  b1.reshape(1, D), W2)
    gp = jnp.pad(g.reshape(-1), (0, NACC + 16 - N))
    dinvp = jnp.pad(dinv.reshape(-1), (0, NACC + 16 - N))
    b2w = jnp.broadcast_to(b2, (16,))
    out = _scal_call(gp, dinvp, b2w, srcw2, dstw2)
    return out[:N]


# speedup vs baseline: 55.5071x; 1.0443x over previous
import jax
import jax.numpy as jnp
from jax import lax
from jax.experimental import pallas as pl
from jax.experimental.pallas import tpu as pltpu
from jax.experimental.pallas import tpu_sc as plsc

N = 10000
D = 128
NC = 2
NS = 16
B = 128
NB = 80
EW = NB * B
EPAD = NC * NS * EW
NB2 = NC * NB
ND = 4
NBQ = NB2 // ND
DH = D // 2
RZ = 632
NACC = NS * RZ
BM = 2000
EDGES = 320000
PAD = EPAD - EDGES
F32 = jnp.float32


def _mesh():
    return plsc.VectorSubcoreMesh(core_axis_name="c", subcore_axis_name="s")


def _deg_body(dst_hbm, degp_hbm, dst_v, ones_v, zv, deg_sh, sem):
    c = lax.axis_index("c")
    s = lax.axis_index("s")
    soff = pl.multiple_of(s * RZ, 8)
    woff = pl.multiple_of((c * NS + s) * RZ, 8)
    for k in range((RZ + 8) // 16):
        zv[pl.ds(k * 16, 16)] = jnp.zeros((16,), F32)
    pltpu.sync_copy(zv.at[pl.ds(0, RZ)], deg_sh.at[pl.ds(soff, RZ)])
    pltpu.sync_copy(dst_hbm.at[c, s], dst_v)
    for k in range(B // 16):
        ones_v[pl.ds(k * 16, 16)] = jnp.ones((16,), F32)
    plsc.subcore_barrier()

    def body(j8, carry):
        j0 = j8 * 8
        for k in range(8):
            pltpu.async_copy(ones_v, deg_sh.at[dst_v.at[j0 + k]], sem,
                             add=True)
        for k in range(8):
            pltpu.make_async_copy(
                ones_v, deg_sh.at[dst_v.at[j0 + k]], sem).wait()
        return carry

    lax.fori_loop(0, NB // 8, body, 0)
    plsc.subcore_barrier()
    pltpu.sync_copy(deg_sh.at[pl.ds(soff, RZ)], zv.at[pl.ds(0, RZ)])
    pltpu.sync_copy(zv.at[pl.ds(0, RZ)], degp_hbm.at[pl.ds(woff, RZ)])


def _deg_call(dstw):
    fn = pl.kernel(
        _deg_body,
        out_type=jax.ShapeDtypeStruct((NC * NACC,), F32),
        mesh=_mesh(),
        scratch_types=[
            pltpu.VMEM((NB, B), jnp.int32),
            pltpu.VMEM((B,), F32),
            pltpu.VMEM((RZ + 8,), F32),
            pltpu.VMEM_SHARED((NACC,), F32),
            pltpu.SemaphoreType.DMA,
        ],
    )
    return fn(dstw)


def _rows_body(h2_hbm, src2_hbm, dst_hbm, outp_hbm,
               src_v, dst_v, rows_v, cp_v, acc_sh, *sems):
    gsems = sems[:ND]
    ssems = sems[ND:]
    c = lax.axis_index("c")
    s = lax.axis_index("s")
    soff = pl.multiple_of(s * RZ, 8)
    woff = pl.multiple_of((c * NS + s) * RZ, 8)

    def zfill(r, carry):
        for k in range(DH // 16):
            cp_v[r, pl.ds(k * 16, 16)] = jnp.zeros((16,), F32)
        return carry

    lax.fori_loop(0, 200, zfill, 0)
    for q, sz in ((0, 200), (200, 200), (400, 200), (600, 32)):
        pltpu.sync_copy(cp_v.at[pl.ds(0, sz)],
                        acc_sh.at[pl.ds(soff + q, sz)])
    pltpu.sync_copy(src2_hbm.at[s], src_v)
    pltpu.sync_copy(dst_hbm.at[s], dst_v)
    plsc.subcore_barrier()

    cvec = jnp.zeros((16,), jnp.int32) + c

    def to_table_idx(j):
        for t in range(B // 16):
            sl = pl.ds(t * 16, 16)
            src_v[j, sl] = 2 * src_v[j, sl] + cvec

    for k in range(ND):
        to_table_idx(k)
        pltpu.async_copy(h2_hbm.at[src_v.at[k]], rows_v.at[k], gsems[k])

    def body(q, carry):
        j0 = q * ND
        for k in range(ND):
            j = j0 + k
            pltpu.make_async_copy(
                h2_hbm.at[src_v.at[j]], rows_v.at[k], gsems[k]).wait()
            pltpu.async_copy(rows_v.at[k], acc_sh.at[dst_v.at[j]],
                             ssems[k], add=True)

            @pl.when(q < NBQ - 1)
            def _():
                to_table_idx(j + ND)
                pltpu.make_async_copy(
                    rows_v.at[k], acc_sh.at[dst_v.at[j]], ssems[k]).wait()
                pltpu.async_copy(
                    h2_hbm.at[src_v.at[j + ND]], rows_v.at[k], gsems[k])

        return carry

    lax.fori_loop(0, NBQ, body, 0)
    for k in range(ND):
        pltpu.make_async_copy(
            rows_v.at[k], acc_sh.at[dst_v.at[NB2 - ND + k]], ssems[k]).wait()
    plsc.subcore_barrier()
    for q, sz in ((0, 200), (200, 200), (400, 200), (600, 32)):
        pltpu.sync_copy(acc_sh.at[pl.ds(soff + q, sz)], cp_v.at[pl.ds(0, sz)])
        pltpu.sync_copy(cp_v.at[pl.ds(0, sz)], outp_hbm.at[pl.ds(woff + q, sz)])


def _rows_call(h2, src2w, dstw2):
    fn = pl.kernel(
        _rows_body,
        out_type=jax.ShapeDtypeStruct((NC * NACC, DH), F32),
        mesh=_mesh(),
        scratch_types=[
            pltpu.VMEM((NB2, B), jnp.int32),
            pltpu.VMEM((NB2, B), jnp.int32),
            pltpu.VMEM((ND, B, DH), F32),
            pltpu.VMEM((200, DH), F32),
            pltpu.VMEM_SHARED((NACC, DH), F32),
        ] + [pltpu.SemaphoreType.DMA] * (2 * ND),
        compiler_params=pltpu.CompilerParams(use_tc_tiling_on_sc=False,
                                             needs_layout_passes=False),
    )
    return fn(h2, src2w, dstw2)


def _scal_body(g_hbm, dinv_hbm, b2_hbm, src_hbm, dst_hbm, out_hbm,
               g_v, dinv_v, b2_v, src_v, dst_v, vals_v, zv, acc_sh,
               g_sh, *sems):
    c = lax.axis_index("c")
    s = lax.axis_index("s")

    @pl.when(c == 0)
    def _():
        soff = pl.multiple_of(s * RZ, 8)
        for k in range((RZ + 8) // 16):
            zv[pl.ds(k * 16, 16)] = jnp.zeros((16,), F32)
        pltpu.sync_copy(zv.at[pl.ds(0, RZ)], acc_sh.at[pl.ds(soff, RZ)])
        pltpu.sync_copy(g_hbm.at[pl.ds(soff, RZ)], zv.at[pl.ds(0, RZ)])
        pltpu.sync_copy(zv.at[pl.ds(0, RZ)], g_sh.at[pl.ds(soff, RZ)])

        @pl.when(s == 0)
        def _():
            pltpu.sync_copy(g_hbm.at[pl.ds(NACC, 16)], b2_v)
            pltpu.sync_copy(b2_v, g_sh.at[pl.ds(NACC, 16)])

        pltpu.sync_copy(dinv_hbm.at[pl.ds(soff, RZ + 16)], dinv_v)
        pltpu.sync_copy(b2_hbm, b2_v)
        pltpu.sync_copy(src_hbm.at[s], src_v)
        pltpu.sync_copy(dst_hbm.at[s], dst_v)
        plsc.subcore_barrier()
        pltpu.sync_copy(g_sh, g_v)

        def body(q, carry):
            j0 = q * 8
            for par in range(8):
                j = j0 + par
                vb = vals_v.at[par]

                @pl.when(q > 0)
                def _():
                    pltpu.make_async_copy(
                        vb, acc_sh.at[dst_v.at[j - 8]], sems[par]).wait()

                for k in range(B // 16):
                    idx = src_v[j, pl.ds(k * 16, 16)]
                    vb[pl.ds(k * 16, 16)] = plsc.load_gather(g_v, [idx])
                pltpu.async_copy(vb, acc_sh.at[dst_v.at[j]], sems[par],
                                 add=True)
            return carry

        lax.fori_loop(0, NB2 // 8, body, 0)
        for par in range(8):
            pltpu.make_async_copy(
                vals_v.at[par], acc_sh.at[dst_v.at[NB2 - 8 + par]],
                sems[par]).wait()
        plsc.subcore_barrier()

        pltpu.sync_copy(acc_sh.at[pl.ds(soff, RZ)], zv.at[pl.ds(0, RZ)])
        b2s = b2_v[pl.ds(0, 16)]
        for k in range((RZ + 8) // 16):
            o = pl.ds(k * 16, 16)
            so = pl.ds(soff + k * 16, 16)
            zv[o] = dinv_v[o] * (zv[o] + g_v[so]) + b2s
        pltpu.sync_copy(zv.at[pl.ds(0, RZ)], out_hbm.at[pl.ds(soff, RZ)])


def _scal_call(g, dinv, b2w, srcw2, dstw2):
    fn = pl.kernel(
        _scal_body,
        out_type=jax.ShapeDtypeStruct((NACC,), F32),
        mesh=_mesh(),
        scratch_types=[
            pltpu.VMEM((NACC + 16,), F32),
            pltpu.VMEM((RZ + 16,), F32),
            pltpu.VMEM((16,), F32),
            pltpu.VMEM((NB2, B), jnp.int32),
            pltpu.VMEM((NB2, B), jnp.int32),
            pltpu.VMEM((8, B), F32),
            pltpu.VMEM((RZ + 8,), F32),
            pltpu.VMEM_SHARED((NACC,), F32),
            pltpu.VMEM_SHARED((NACC + 16,), F32),
        ] + [pltpu.SemaphoreType.DMA] * 8,
        compiler_params=pltpu.CompilerParams(needs_layout_passes=False),
    )
    return fn(g, dinv, b2w, srcw2, dstw2)


def _prep_body(ei_ref, srcp_ref, dstp_ref):
    pr = EPAD // B - (EPAD - PAD) // B
    pidr = lax.broadcasted_iota(jnp.int32, (pr, B), 0) * B + \
        lax.broadcasted_iota(jnp.int32, (pr, B), 1)
    srcp_ref[...] = jnp.concatenate([ei_ref[0], pidr], axis=0)
    dstp_ref[...] = jnp.concatenate(
        [ei_ref[1], N + (pidr & 63)], axis=0)


def _prep_call(ei3):
    return pl.pallas_call(
        _prep_body,
        grid=(1,),
        in_specs=[pl.BlockSpec((2, (EPAD - PAD) // B, B),
                               lambda i: (0, 0, 0))],
        out_specs=[
            pl.BlockSpec((EPAD // B, B), lambda i: (0, 0)),
            pl.BlockSpec((EPAD // B, B), lambda i: (0, 0)),
        ],
        out_shape=[
            jax.ShapeDtypeStruct((EPAD // B, B), jnp.int32),
            jax.ShapeDtypeStruct((EPAD // B, B), jnp.int32),
        ],
    )(ei3)


def _mm1_body(x_ref, w_ref, degp_ref, h_ref, dinv_ref):
    deg = degp_ref[0] + degp_ref[1] + 1.0
    dinv = lax.rsqrt(deg)
    h = jnp.dot(x_ref[...], w_ref[...], preferred_element_type=F32)
    h_ref[...] = h * dinv
    dinv_ref[...] = dinv


def _mm1_call(x, W1, degp3):
    return pl.pallas_call(
        _mm1_body,
        grid=(N // BM,),
        in_specs=[
            pl.BlockSpec((BM, D), lambda i: (i, 0)),
            pl.BlockSpec((D, D), lambda i: (0, 0)),
            pl.BlockSpec((NC, BM, 1), lambda i: (0, i, 0)),
        ],
        out_specs=[
            pl.BlockSpec((BM, D), lambda i: (i, 0)),
            pl.BlockSpec((BM, 1), lambda i: (i, 0)),
        ],
        out_shape=[
            jax.ShapeDtypeStruct((N, D), F32),
            jax.ShapeDtypeStruct((N, 1), F32),
        ],
    )(x, W1, degp3)


def _mm2_body(p_ref, h_ref, dinv_ref, b1_ref, w2_ref, g_ref):
    ps = jnp.concatenate([p_ref[0], p_ref[1]], axis=-1) + h_ref[...]
    o1 = ps * dinv_ref[...] + b1_ref[...]
    r = jnp.maximum(o1, 0.0)
    g = jnp.dot(r, w2_ref[...], preferred_element_type=F32)
    g_ref[...] = g * dinv_ref[...]


def _mm2_call(p, h, dinv, b1r, W2):
    return pl.pallas_call(
        _mm2_body,
        grid=(N // BM,),
        in_specs=[
            pl.BlockSpec((NC, BM, DH), lambda i: (0, i, 0)),
            pl.BlockSpec((BM, D), lambda i: (i, 0)),
            pl.BlockSpec((BM, 1), lambda i: (i, 0)),
            pl.BlockSpec((1, D), lambda i: (0, 0)),
            pl.BlockSpec((D, 1), lambda i: (0, 0)),
        ],
        out_specs=pl.BlockSpec((BM, 1), lambda i: (i, 0)),
        out_shape=jax.ShapeDtypeStruct((N, 1), F32),
    )(p, h, dinv, b1r, W2)


def kernel(x, edge_index, W1, b1, W2, b2):
    ei = edge_index.astype(jnp.int32)
    srcp2, dstp2 = _prep_call(ei.reshape(2, EDGES // B, B))
    srcw2 = srcp2.reshape(NS, NB2, B)
    dstw = dstp2.reshape(NC, NS, NB, B)
    dstw2 = dstp2.reshape(NS, NB2, B)

    degp = _deg_call(dstw)
    h, dinv = _mm1_call(x, W1, degp.reshape(NC, NACC, 1))
    p = _rows_call(h.reshape(NC * N, DH), srcw2, dstw2)
    g = _mm2_call(p.reshape(NC, NACC, DH), h, dinv,
                  b1.reshape(1, D), W2)
    gp = jnp.pad(g.reshape(-1), (0, NACC + 16 - N))
    dinvp = jnp.pad(dinv.reshape(-1), (0, NACC + 16 - N))
    b2w = jnp.broadcast_to(b2, (16,))
    out = _scal_call(gp, dinvp, b2w, srcw2, dstw2)
    return out[:N]
